# baseline (speedup 1.0000x reference)
import jax
import jax.numpy as jnp
from jax.experimental import pallas as pl

DIM = 64
N_EMBED = 8192
TOKENS = 32 * 1024
BT = 256
NBLK = TOKENS // BT


def _vq_body(x_ref, e_ref, ind_ref, q_ref, loss_ref):
    i = pl.program_id(0)
    x = x_ref[...]
    e = e_ref[...]
    x_sq = jnp.sum(x * x, axis=1, keepdims=True)
    e_sq = jnp.sum(e * e, axis=0, keepdims=True)
    lhs = (2.0 * x).astype(jnp.bfloat16)
    e_hi = e.astype(jnp.bfloat16)
    mm2 = jnp.dot(lhs, e_hi, preferred_element_type=jnp.float32)
    dist = (x_sq - mm2) + e_sq

    min_val = jnp.min(dist, axis=1, keepdims=True)
    col = jax.lax.broadcasted_iota(jnp.int32, dist.shape, 1)
    ind = jnp.min(jnp.where(dist == min_val, col, N_EMBED), axis=1)
    ind_ref[0, 0, :] = ind.astype(jnp.int32)

    onehot = (col == ind[:, None]).astype(jnp.bfloat16)
    e_lo = (e - e_hi.astype(jnp.float32)).astype(jnp.bfloat16)
    dn = (((1,), (1,)), ((), ()))
    q = (jax.lax.dot_general(onehot, e_hi, dn, preferred_element_type=jnp.float32)
         + jax.lax.dot_general(onehot, e_lo, dn, preferred_element_type=jnp.float32))
    q_ref[...] = q

    d = q - x
    part = jnp.sum(d * d).reshape(1, 1)

    @pl.when(i == 0)
    def _():
        loss_ref[...] = jnp.zeros((1, 1), jnp.float32)

    loss_ref[...] += part


def kernel(input, embed):
    flat = input.reshape(TOKENS, DIM)
    ind3, q, loss_sum = pl.pallas_call(
        _vq_body,
        grid=(NBLK,),
        in_specs=[
            pl.BlockSpec((BT, DIM), lambda i: (i, 0)),
            pl.BlockSpec((DIM, N_EMBED), lambda i: (0, 0)),
        ],
        out_specs=[
            pl.BlockSpec((1, 1, BT), lambda i: (i, 0, 0)),
            pl.BlockSpec((BT, DIM), lambda i: (i, 0)),
            pl.BlockSpec((1, 1), lambda i: (0, 0)),
        ],
        out_shape=[
            jax.ShapeDtypeStruct((NBLK, 1, BT), jnp.int32),
            jax.ShapeDtypeStruct((TOKENS, DIM), jnp.float32),
            jax.ShapeDtypeStruct((1, 1), jnp.float32),
        ],
    )(flat, embed)
    embed_ind = ind3.reshape(input.shape[:-1])
    quantize = q.reshape(input.shape)
    loss = loss_sum[0, 0] / jnp.float32(TOKENS * DIM)
    return (quantize, embed_ind, loss)
